# Initial kernel scaffold; baseline (speedup 1.0000x reference)
import jax
import jax.numpy as jnp
from jax.experimental import pallas as pl


def kernel(fixed_tensor, tables):
    raise NotImplementedError("write your pallas kernel here")


# speedup vs baseline: 1.1514x; 1.1514x over previous
import functools

import jax
import jax.numpy as jnp
from jax import lax
from jax.experimental import pallas as pl
from jax.experimental.pallas import tpu as pltpu
from jax.experimental.pallas import tpu_sc as plsc

_NUM_FIELDS = 26
_VOCAB = 100000
_EMB_DIM = 16
_BATCH = 16384

try:
    _info = plsc.get_sparse_core_info()
    _NC, _NS, _L = _info.num_cores, _info.num_subcores, _info.num_lanes
except Exception:
    _NC, _NS, _L = 2, 16, 16

_NW = _NC * _NS
_TOTAL = _BATCH * _NUM_FIELDS
_RPW = _TOTAL // _NW
_NCH = 8
_CH = _RPW // _NCH
_VPC = _CH // _L


def _make_sc_gather():
    mesh = plsc.VectorSubcoreMesh(core_axis_name="c", subcore_axis_name="s")

    @functools.partial(
        pl.kernel,
        out_type=jax.ShapeDtypeStruct((_TOTAL, _EMB_DIM), jnp.float32),
        mesh=mesh,
        compiler_params=pltpu.CompilerParams(use_tc_tiling_on_sc=False),
        scratch_types=(
            [pltpu.VMEM((_CH,), jnp.int32) for _ in range(_NCH)]
            + [
                pltpu.VMEM((_NCH, _CH), jnp.int32),
                pltpu.VMEM((_CH, _EMB_DIM), jnp.float32),
                pltpu.VMEM((_CH, _EMB_DIM), jnp.float32),
                pltpu.SemaphoreType.DMA,
                pltpu.SemaphoreType.DMA,
            ]
        ),
    )
    def k(tab_hbm, idx_hbm, off_hbm, out_hbm, *scratch):
        idx_vs = scratch[:_NCH]
        off_v, buf0, buf1, sem0, sem1 = scratch[_NCH:]
        wid = lax.axis_index("s") * _NC + lax.axis_index("c")
        chunk0 = wid * _NCH
        row0 = wid * _RPW

        for c in range(_NCH):
            pltpu.sync_copy(idx_hbm.at[chunk0 + c], idx_vs[c])
        pltpu.sync_copy(off_hbm, off_v)

        bufs = (buf0, buf1)
        sems = (sem0, sem1)

        def add_offsets(c):
            def body(j, _):
                s = pl.ds(j * _L, _L)
                idx_vs[c][s] = idx_vs[c][s] + off_v[c, s]
                return 0
            lax.fori_loop(0, _VPC, body, 0)

        def fire(c):
            return pltpu.async_copy(
                tab_hbm.at[idx_vs[c]], bufs[c % 2], sems[c % 2])

        def store(c):
            pltpu.sync_copy(
                bufs[c % 2], out_hbm.at[pl.ds(row0 + c * _CH, _CH)])

        add_offsets(0)
        descs = [fire(0)]
        for c in range(1, _NCH):
            add_offsets(c)
            descs.append(fire(c))
            descs[c - 1].wait()
            store(c - 1)
        descs[_NCH - 1].wait()
        store(_NCH - 1)

    return k


_sc_gather = _make_sc_gather()


def kernel(fixed_tensor, tables):
    idx2 = fixed_tensor.astype(jnp.int32).reshape(_NW * _NCH, _CH)
    tab_flat = tables.reshape(_NUM_FIELDS * _VOCAB, _EMB_DIM)
    off2 = ((jnp.arange(_RPW, dtype=jnp.int32) % _NUM_FIELDS) * _VOCAB
            ).reshape(_NCH, _CH)
    out_flat = _sc_gather(tab_flat, idx2, off2)
    return out_flat.reshape(_BATCH, _NUM_FIELDS * _EMB_DIM)
